# baseline (speedup 1.0000x reference)
import functools

import jax
import jax.numpy as jnp
from jax import lax
from jax.experimental import pallas as pl
from jax.experimental.pallas import tpu as pltpu
from jax.experimental.pallas import tpu_sc as plsc

VOCAB = 1000000
DIM = 64
BATCH = 16384
NNEG = 20

NC = 2
NS = 16
NW = NC * NS
BPW = BATCH // NW
CB = 32
NCHUNK = BPW // CB
NEG_SEG = 128
NEG_PER_CHUNK = CB * NNEG
NEG_DMAS = NEG_PER_CHUNK // NEG_SEG
PDIM = 2 * DIM


def _sc_scores_body(cenp, ctxp, negp, cenc, ctxc, negc, cen_tab, ctx_tab, out,
                    idxc, idxx, idxn, ccolb, xcolb, ncolb, cenv, ctxv, negv,
                    score, sem):
  wid = lax.axis_index("s") * NC + lax.axis_index("c")
  base = wid * BPW

  pltpu.sync_copy(cenp.at[pl.ds(base, BPW)], idxc)
  pltpu.sync_copy(ctxp.at[pl.ds(base, BPW)], idxx)
  pltpu.sync_copy(negp.at[pl.ds(base * NNEG, BPW * NNEG)], idxn)
  pltpu.sync_copy(cenc.at[pl.ds(base, BPW)], ccolb)
  pltpu.sync_copy(ctxc.at[pl.ds(base, BPW)], xcolb)
  pltpu.sync_copy(negc.at[:, pl.ds(base, BPW)], ncolb)

  iota16 = lax.iota(jnp.int32, 16)
  lane15 = iota16 == 15
  nsplat = [jnp.full((16,), n, jnp.int32) for n in range(NNEG + 1)]

  def chunk(i, _):
    copies = [
        pltpu.async_copy(cen_tab.at[idxc.at[pl.ds(i * CB, CB)]], cenv, sem),
        pltpu.async_copy(ctx_tab.at[idxx.at[pl.ds(i * CB, CB)]], ctxv, sem),
    ]
    for j in range(NEG_DMAS):
      copies.append(
          pltpu.async_copy(
              ctx_tab.at[idxn.at[pl.ds(i * NEG_PER_CHUNK + j * NEG_SEG,
                                       NEG_SEG)]],
              negv.at[pl.ds(j * NEG_SEG, NEG_SEG)], sem))
    for c in copies:
      c.wait()

    def body(b, _):
      brow = jnp.broadcast_to(b, (16,)).astype(jnp.int32)
      col = jnp.broadcast_to(i * CB + b, (16,))

      ccol = plsc.load_gather(ccolb, [col]) + iota16
      xcol = plsc.load_gather(xcolb, [col]) + iota16
      cs = [plsc.load_gather(cenv, [brow, ccol + 16 * k]) for k in range(4)]
      xs = [plsc.load_gather(ctxv, [brow, xcol + 16 * k]) for k in range(4)]
      p = cs[0] * xs[0] + cs[1] * xs[1] + cs[2] * xs[2] + cs[3] * xs[3]
      plsc.store_scatter(score, [nsplat[NNEG], col], plsc.cumsum(p),
                         mask=lane15)
      for n in range(NNEG):
        nrow = jnp.broadcast_to(b * NNEG + n, (16,)).astype(jnp.int32)
        gcol = plsc.load_gather(ncolb, [nsplat[n], col]) + iota16
        gs = [plsc.load_gather(negv, [nrow, gcol + 16 * k]) for k in range(4)]
        q = cs[0] * gs[0] + cs[1] * gs[1] + cs[2] * gs[2] + cs[3] * gs[3]
        plsc.store_scatter(score, [nsplat[n], col], -plsc.cumsum(q),
                           mask=lane15)
      return 0

    lax.fori_loop(0, CB, body, 0)
    return 0

  lax.fori_loop(0, NCHUNK, chunk, 0)

  pltpu.sync_copy(score, out.at[pl.ds(wid * (NNEG + 1), NNEG + 1)])


_sc_scores = functools.partial(
    pl.kernel,
    out_type=jax.ShapeDtypeStruct((NW * (NNEG + 1), BPW), jnp.float32),
    mesh=plsc.VectorSubcoreMesh(core_axis_name="c", subcore_axis_name="s"),
    compiler_params=pltpu.CompilerParams(
        needs_layout_passes=False, use_tc_tiling_on_sc=False),
    scratch_types=[
        pltpu.VMEM((BPW,), jnp.int32),
        pltpu.VMEM((BPW,), jnp.int32),
        pltpu.VMEM((BPW * NNEG,), jnp.int32),
        pltpu.VMEM((BPW,), jnp.int32),
        pltpu.VMEM((BPW,), jnp.int32),
        pltpu.VMEM((NNEG, BPW), jnp.int32),
        pltpu.VMEM((CB, PDIM), jnp.float32),
        pltpu.VMEM((CB, PDIM), jnp.float32),
        pltpu.VMEM((NEG_PER_CHUNK, PDIM), jnp.float32),
        pltpu.VMEM((NNEG + 1, BPW), jnp.float32),
        pltpu.SemaphoreType.DMA,
    ],
)(_sc_scores_body)


def _loss_body(s_ref, o_ref):
  x = s_ref[...]
  y = jnp.minimum(x, 0.0) - jnp.log1p(jnp.exp(-jnp.abs(x)))
  o_ref[0, 0] = -jnp.sum(y) * (1.0 / BATCH)


_loss = pl.pallas_call(
    _loss_body,
    out_shape=jax.ShapeDtypeStruct((1, 1), jnp.float32),
    out_specs=pl.BlockSpec(memory_space=pltpu.SMEM),
)


def kernel(center_ids, context_ids, negative_ids, center_table, context_table):
  cen_ids = center_ids.astype(jnp.int32)
  ctx_ids = context_ids.astype(jnp.int32)
  neg_ids = negative_ids.astype(jnp.int32)
  scores = _sc_scores(
      cen_ids >> 1, ctx_ids >> 1, (neg_ids >> 1).reshape(BATCH * NNEG),
      (cen_ids & 1) << 6, (ctx_ids & 1) << 6, ((neg_ids & 1) << 6).T,
      center_table.reshape(VOCAB // 2, PDIM),
      context_table.reshape(VOCAB // 2, PDIM))
  return _loss(scores)[0, 0]


# speedup vs baseline: 1.7122x; 1.7122x over previous
import functools

import jax
import jax.numpy as jnp
from jax import lax
from jax.experimental import pallas as pl
from jax.experimental.pallas import tpu as pltpu
from jax.experimental.pallas import tpu_sc as plsc

VOCAB = 1000000
DIM = 64
BATCH = 16384
NNEG = 20

NC = 2
NS = 16
NW = NC * NS
BPW = BATCH // NW
CB = 32
NCHUNK = BPW // CB
NEG_SEG = 128
NEG_PER_CHUNK = CB * NNEG
NEG_DMAS = NEG_PER_CHUNK // NEG_SEG
PDIM = 2 * DIM


def _sc_scores_body(cenp, ctxp, negp, cenc, ctxc, negc, cen_tab, ctx_tab, out,
                    idxc, idxx, idxn, ccolb, xcolb, ncolb, cenv, ctxv, negv,
                    score, sem):
  wid = lax.axis_index("s") * NC + lax.axis_index("c")
  base = wid * BPW

  pltpu.sync_copy(cenp.at[pl.ds(base, BPW)], idxc)
  pltpu.sync_copy(ctxp.at[pl.ds(base, BPW)], idxx)
  pltpu.sync_copy(negp.at[pl.ds(base * NNEG, BPW * NNEG)], idxn)
  pltpu.sync_copy(cenc.at[pl.ds(base, BPW)], ccolb)
  pltpu.sync_copy(ctxc.at[pl.ds(base, BPW)], xcolb)
  pltpu.sync_copy(negc.at[:, pl.ds(base, BPW)], ncolb)

  iota16 = lax.iota(jnp.int32, 16)
  lane15 = iota16 == 15
  nsplat = [jnp.full((16,), n, jnp.int32) for n in range(NNEG + 1)]

  def chunk(i, _):
    copies = [
        pltpu.async_copy(cen_tab.at[idxc.at[pl.ds(i * CB, CB)]], cenv, sem),
        pltpu.async_copy(ctx_tab.at[idxx.at[pl.ds(i * CB, CB)]], ctxv, sem),
    ]
    for j in range(NEG_DMAS):
      copies.append(
          pltpu.async_copy(
              ctx_tab.at[idxn.at[pl.ds(i * NEG_PER_CHUNK + j * NEG_SEG,
                                       NEG_SEG)]],
              negv.at[pl.ds(j * NEG_SEG, NEG_SEG)], sem))
    for c in copies:
      c.wait()

    def body(b, _):
      brow = jnp.broadcast_to(b, (16,)).astype(jnp.int32)
      col = jnp.broadcast_to(i * CB + b, (16,))

      ccol = plsc.load_gather(ccolb, [col]) + iota16
      xcol = plsc.load_gather(xcolb, [col]) + iota16
      cs = [plsc.load_gather(cenv, [brow, ccol + 16 * k]) for k in range(4)]
      xs = [plsc.load_gather(ctxv, [brow, xcol + 16 * k]) for k in range(4)]
      p = cs[0] * xs[0] + cs[1] * xs[1] + cs[2] * xs[2] + cs[3] * xs[3]
      plsc.store_scatter(score, [nsplat[NNEG], col], plsc.cumsum(p),
                         mask=lane15)
      for n in range(NNEG):
        nrow = jnp.broadcast_to(b * NNEG + n, (16,)).astype(jnp.int32)
        gcol = plsc.load_gather(ncolb, [nsplat[n], col]) + iota16
        gs = [plsc.load_gather(negv, [nrow, gcol + 16 * k]) for k in range(4)]
        q = cs[0] * gs[0] + cs[1] * gs[1] + cs[2] * gs[2] + cs[3] * gs[3]
        plsc.store_scatter(score, [nsplat[n], col], -plsc.cumsum(q),
                           mask=lane15)
      return 0

    lax.fori_loop(0, CB, body, 0)
    return 0

  lax.fori_loop(0, NCHUNK, chunk, 0)

  pltpu.sync_copy(score, out.at[pl.ds(wid * (NNEG + 1), NNEG + 1)])


_sc_scores = functools.partial(
    pl.kernel,
    out_type=jax.ShapeDtypeStruct((NW * (NNEG + 1), BPW), jnp.float32),
    mesh=plsc.VectorSubcoreMesh(core_axis_name="c", subcore_axis_name="s"),
    compiler_params=pltpu.CompilerParams(
        needs_layout_passes=False, use_tc_tiling_on_sc=False),
    scratch_types=[
        pltpu.VMEM((BPW,), jnp.int32),
        pltpu.VMEM((BPW,), jnp.int32),
        pltpu.VMEM((BPW * NNEG,), jnp.int32),
        pltpu.VMEM((BPW,), jnp.int32),
        pltpu.VMEM((BPW,), jnp.int32),
        pltpu.VMEM((NNEG, BPW), jnp.int32),
        pltpu.VMEM((CB, PDIM), jnp.float32),
        pltpu.VMEM((CB, PDIM), jnp.float32),
        pltpu.VMEM((NEG_PER_CHUNK, PDIM), jnp.float32),
        pltpu.VMEM((NNEG + 1, BPW), jnp.float32),
        pltpu.SemaphoreType.DMA,
    ],
)(_sc_scores_body)


TBK = 6400
NTBK = 79
PROWS = NTBK * TBK


def _transpose_body(a_ref, b_ref, o_ref):
  eye = (lax.broadcasted_iota(jnp.int32, (DIM, DIM), 0) ==
         lax.broadcasted_iota(jnp.int32, (DIM, DIM), 1)).astype(jnp.float32)
  o_ref[:, 0:DIM] = lax.dot_general(a_ref[...], eye, (((0,), (0,)), ((), ())))
  o_ref[:, DIM:2 * DIM] = lax.dot_general(b_ref[...], eye,
                                          (((0,), (0,)), ((), ())))


_repack = pl.pallas_call(
    _transpose_body,
    grid=(NTBK,),
    in_specs=[
        pl.BlockSpec((DIM, TBK), lambda g: (0, 2 * g)),
        pl.BlockSpec((DIM, TBK),
                     lambda g: (0, jnp.minimum(2 * g + 1, VOCAB // TBK))),
    ],
    out_specs=pl.BlockSpec((TBK, PDIM), lambda g: (g, 0)),
    out_shape=jax.ShapeDtypeStruct((PROWS, PDIM), jnp.float32),
)


def _repack_table(table):
  tt = table.T
  return _repack(tt, tt)


def _loss_body(s_ref, o_ref):
  x = s_ref[...]
  y = jnp.minimum(x, 0.0) - jnp.log1p(jnp.exp(-jnp.abs(x)))
  o_ref[0, 0] = -jnp.sum(y) * (1.0 / BATCH)


_loss = pl.pallas_call(
    _loss_body,
    out_shape=jax.ShapeDtypeStruct((1, 1), jnp.float32),
    out_specs=pl.BlockSpec(memory_space=pltpu.SMEM),
)


def kernel(center_ids, context_ids, negative_ids, center_table, context_table):
  cen_ids = center_ids.astype(jnp.int32)
  ctx_ids = context_ids.astype(jnp.int32)
  neg_ids = negative_ids.astype(jnp.int32)

  def row(ids):
    blk = ids // TBK
    return (blk >> 1) * TBK + ids % TBK

  def colb(ids):
    return ((ids // TBK) & 1) << 6

  scores = _sc_scores(
      row(cen_ids), row(ctx_ids), row(neg_ids).reshape(BATCH * NNEG),
      colb(cen_ids), colb(ctx_ids), colb(neg_ids).T,
      _repack_table(center_table), _repack_table(context_table))
  return _loss(scores)[0, 0]


# speedup vs baseline: 1.8600x; 1.0863x over previous
import functools

import jax
import jax.numpy as jnp
from jax import lax
from jax.experimental import pallas as pl
from jax.experimental.pallas import tpu as pltpu
from jax.experimental.pallas import tpu_sc as plsc

VOCAB = 1000000
DIM = 64
BATCH = 16384
NNEG = 20

NC = 2
NS = 16
NW = NC * NS
BPW = BATCH // NW
CB = 16
NCHUNK = BPW // CB
NEG_SEG = 64
NEG_PER_CHUNK = CB * NNEG
NEG_DMAS = NEG_PER_CHUNK // NEG_SEG
PDIM = 2 * DIM


def _sc_scores_body(cenp, ctxp, negp, cenc, ctxc, negc, cen_tab, ctx_tab, out,
                    idxc, idxx, idxn, ccolb, xcolb, ncolb, cenv0, ctxv0,
                    negv0, cenv1, ctxv1, negv1, score, sem0, sem1):
  wid = lax.axis_index("s") * NC + lax.axis_index("c")
  base = wid * BPW

  pltpu.sync_copy(cenp.at[pl.ds(base, BPW)], idxc)
  pltpu.sync_copy(ctxp.at[pl.ds(base, BPW)], idxx)
  pltpu.sync_copy(negp.at[pl.ds(base * NNEG, BPW * NNEG)], idxn)
  pltpu.sync_copy(cenc.at[pl.ds(base, BPW)], ccolb)
  pltpu.sync_copy(ctxc.at[pl.ds(base, BPW)], xcolb)
  pltpu.sync_copy(negc.at[:, pl.ds(base, BPW)], ncolb)

  iota16 = lax.iota(jnp.int32, 16)
  lane15 = iota16 == 15
  nsplat = [jnp.full((16,), n, jnp.int32) for n in range(NNEG + 1)]
  bufs = ((cenv0, ctxv0, negv0, sem0), (cenv1, ctxv1, negv1, sem1))
  last = NCHUNK - 1

  def descs(i, buf):
    cenv, ctxv, negv, sem = buf
    ds = [
        pltpu.make_async_copy(cen_tab.at[idxc.at[pl.ds(i * CB, CB)]],
                              cenv, sem),
        pltpu.make_async_copy(ctx_tab.at[idxx.at[pl.ds(i * CB, CB)]],
                              ctxv, sem),
    ]
    for j in range(NEG_DMAS):
      ds.append(
          pltpu.make_async_copy(
              ctx_tab.at[idxn.at[pl.ds(i * NEG_PER_CHUNK + j * NEG_SEG,
                                       NEG_SEG)]],
              negv.at[pl.ds(j * NEG_SEG, NEG_SEG)], sem))
    return ds

  def fire(i, buf):
    for d in descs(i, buf):
      d.start()

  def drain(i, buf):
    for d in descs(i, buf):
      d.wait()

  def compute(i, buf):
    cenv, ctxv, negv, _ = buf

    def body(b, _):
      brow = jnp.broadcast_to(b, (16,)).astype(jnp.int32)
      col = jnp.broadcast_to(i * CB + b, (16,))

      ccol = plsc.load_gather(ccolb, [col]) + iota16
      xcol = plsc.load_gather(xcolb, [col]) + iota16
      cs = [plsc.load_gather(cenv, [brow, ccol + 16 * k]) for k in range(4)]
      xs = [plsc.load_gather(ctxv, [brow, xcol + 16 * k]) for k in range(4)]
      p = cs[0] * xs[0] + cs[1] * xs[1] + cs[2] * xs[2] + cs[3] * xs[3]
      plsc.store_scatter(score, [nsplat[NNEG], col], plsc.cumsum(p),
                         mask=lane15)
      for n in range(NNEG):
        nrow = jnp.broadcast_to(b * NNEG + n, (16,)).astype(jnp.int32)
        gcol = plsc.load_gather(ncolb, [nsplat[n], col]) + iota16
        gs = [plsc.load_gather(negv, [nrow, gcol + 16 * k]) for k in range(4)]
        q = cs[0] * gs[0] + cs[1] * gs[1] + cs[2] * gs[2] + cs[3] * gs[3]
        plsc.store_scatter(score, [nsplat[n], col], -plsc.cumsum(q),
                           mask=lane15)
      return 0

    lax.fori_loop(0, CB, body, 0)

  fire(0, bufs[0])

  def pair(ip, _):
    i0 = 2 * ip
    fire(jnp.minimum(i0 + 1, last), bufs[1])
    drain(i0, bufs[0])
    compute(i0, bufs[0])
    fire(jnp.minimum(i0 + 2, last), bufs[0])
    drain(jnp.minimum(i0 + 1, last), bufs[1])
    compute(i0 + 1, bufs[1])
    return 0

  lax.fori_loop(0, NCHUNK // 2, pair, 0)
  drain(last, bufs[0])

  pltpu.sync_copy(score, out.at[pl.ds(wid * (NNEG + 1), NNEG + 1)])


_sc_scores = functools.partial(
    pl.kernel,
    out_type=jax.ShapeDtypeStruct((NW * (NNEG + 1), BPW), jnp.float32),
    mesh=plsc.VectorSubcoreMesh(core_axis_name="c", subcore_axis_name="s"),
    compiler_params=pltpu.CompilerParams(
        needs_layout_passes=False, use_tc_tiling_on_sc=False),
    scratch_types=[
        pltpu.VMEM((BPW,), jnp.int32),
        pltpu.VMEM((BPW,), jnp.int32),
        pltpu.VMEM((BPW * NNEG,), jnp.int32),
        pltpu.VMEM((BPW,), jnp.int32),
        pltpu.VMEM((BPW,), jnp.int32),
        pltpu.VMEM((NNEG, BPW), jnp.int32),
        pltpu.VMEM((CB, PDIM), jnp.float32),
        pltpu.VMEM((CB, PDIM), jnp.float32),
        pltpu.VMEM((NEG_PER_CHUNK, PDIM), jnp.float32),
        pltpu.VMEM((CB, PDIM), jnp.float32),
        pltpu.VMEM((CB, PDIM), jnp.float32),
        pltpu.VMEM((NEG_PER_CHUNK, PDIM), jnp.float32),
        pltpu.VMEM((NNEG + 1, BPW), jnp.float32),
        pltpu.SemaphoreType.DMA,
        pltpu.SemaphoreType.DMA,
    ],
)(_sc_scores_body)


TBK = 6400
NTBK = 79
PROWS = NTBK * TBK


def _transpose_body(a_ref, b_ref, o_ref):
  o_ref[:, 0:DIM] = a_ref[...].T
  o_ref[:, DIM:2 * DIM] = b_ref[...].T


_repack = pl.pallas_call(
    _transpose_body,
    grid=(NTBK,),
    in_specs=[
        pl.BlockSpec((DIM, TBK), lambda g: (0, 2 * g)),
        pl.BlockSpec((DIM, TBK),
                     lambda g: (0, jnp.minimum(2 * g + 1, VOCAB // TBK))),
    ],
    out_specs=pl.BlockSpec((TBK, PDIM), lambda g: (g, 0)),
    out_shape=jax.ShapeDtypeStruct((PROWS, PDIM), jnp.float32),
)


def _repack_table(table):
  tt = table.T
  return _repack(tt, tt)


def _loss_body(s_ref, o_ref):
  x = s_ref[...]
  y = jnp.minimum(x, 0.0) - jnp.log1p(jnp.exp(-jnp.abs(x)))
  o_ref[0, 0] = -jnp.sum(y) * (1.0 / BATCH)


_loss = pl.pallas_call(
    _loss_body,
    out_shape=jax.ShapeDtypeStruct((1, 1), jnp.float32),
    out_specs=pl.BlockSpec(memory_space=pltpu.SMEM),
)


def kernel(center_ids, context_ids, negative_ids, center_table, context_table):
  cen_ids = center_ids.astype(jnp.int32)
  ctx_ids = context_ids.astype(jnp.int32)
  neg_ids = negative_ids.astype(jnp.int32)

  def row(ids):
    blk = ids // TBK
    return (blk >> 1) * TBK + ids % TBK

  def colb(ids):
    return ((ids // TBK) & 1) << 6

  scores = _sc_scores(
      row(cen_ids), row(ctx_ids), row(neg_ids).reshape(BATCH * NNEG),
      colb(cen_ids), colb(ctx_ids), colb(neg_ids).T,
      _repack_table(center_table), _repack_table(context_table))
  return _loss(scores)[0, 0]


# speedup vs baseline: 1.9597x; 1.0536x over previous
import functools

import jax
import jax.numpy as jnp
from jax import lax
from jax.experimental import pallas as pl
from jax.experimental.pallas import tpu as pltpu
from jax.experimental.pallas import tpu_sc as plsc

VOCAB = 1000000
DIM = 64
BATCH = 16384
NNEG = 20

NC = 2
NS = 16
NW = NC * NS
BPW = BATCH // NW
CB = 16
NCHUNK = BPW // CB
NEG_SEG = 64
NEG_PER_CHUNK = CB * NNEG
NEG_DMAS = NEG_PER_CHUNK // NEG_SEG
PDIM = 2 * DIM


def _sc_scores_body(cenp, ctxp, negp, cenc, ctxc, negc, cen_tab, ctx_tab, out,
                    idxc, idxx, idxn, ccolb, xcolb, ncolb, cenv0, ctxv0,
                    negv0, cenv1, ctxv1, negv1, score, sem0, sem1):
  wid = lax.axis_index("s") * NC + lax.axis_index("c")
  base = wid * BPW

  pltpu.sync_copy(cenp.at[pl.ds(base, BPW)], idxc)
  pltpu.sync_copy(ctxp.at[pl.ds(base, BPW)], idxx)
  pltpu.sync_copy(negp.at[pl.ds(base * NNEG, BPW * NNEG)], idxn)
  pltpu.sync_copy(cenc.at[pl.ds(base, BPW)], ccolb)
  pltpu.sync_copy(ctxc.at[pl.ds(base, BPW)], xcolb)
  pltpu.sync_copy(negc.at[:, pl.ds(base, BPW)], ncolb)

  iota16 = lax.iota(jnp.int32, 16)
  lane15 = iota16 == 15
  nsplat = [jnp.full((16,), n, jnp.int32) for n in range(NNEG + 1)]
  bufs = ((cenv0, ctxv0, negv0, sem0), (cenv1, ctxv1, negv1, sem1))
  last = NCHUNK - 1

  def descs(i, buf):
    cenv, ctxv, negv, sem = buf
    ds = [
        pltpu.make_async_copy(cen_tab.at[idxc.at[pl.ds(i * CB, CB)]],
                              cenv, sem),
        pltpu.make_async_copy(ctx_tab.at[idxx.at[pl.ds(i * CB, CB)]],
                              ctxv, sem),
    ]
    for j in range(NEG_DMAS):
      ds.append(
          pltpu.make_async_copy(
              ctx_tab.at[idxn.at[pl.ds(i * NEG_PER_CHUNK + j * NEG_SEG,
                                       NEG_SEG)]],
              negv.at[pl.ds(j * NEG_SEG, NEG_SEG)], sem))
    return ds

  def fire(i, buf):
    for d in descs(i, buf):
      d.start()

  def drain(i, buf):
    for d in descs(i, buf):
      d.wait()

  def compute(i, buf):
    cenv, ctxv, negv, _ = buf

    def body(b, _):
      brow = jnp.broadcast_to(b, (16,)).astype(jnp.int32)
      col = jnp.broadcast_to(i * CB + b, (16,))

      ccol = plsc.load_gather(ccolb, [col]) + iota16
      xcol = plsc.load_gather(xcolb, [col]) + iota16
      cs = [plsc.load_gather(cenv, [brow, ccol + 16 * k]) for k in range(4)]
      xs = [plsc.load_gather(ctxv, [brow, xcol + 16 * k]) for k in range(4)]
      p = cs[0] * xs[0] + cs[1] * xs[1] + cs[2] * xs[2] + cs[3] * xs[3]
      plsc.store_scatter(score, [nsplat[NNEG], col], plsc.cumsum(p),
                         mask=lane15)
      for n in range(NNEG):
        nrow = jnp.broadcast_to(b * NNEG + n, (16,)).astype(jnp.int32)
        gcol = plsc.load_gather(ncolb, [nsplat[n], col]) + iota16
        gs = [plsc.load_gather(negv, [nrow, gcol + 16 * k]) for k in range(4)]
        q = cs[0] * gs[0] + cs[1] * gs[1] + cs[2] * gs[2] + cs[3] * gs[3]
        plsc.store_scatter(score, [nsplat[n], col], -plsc.cumsum(q),
                           mask=lane15)
      return 0

    lax.fori_loop(0, CB, body, 0)

  fire(0, bufs[0])

  def pair(ip, _):
    i0 = 2 * ip
    fire(jnp.minimum(i0 + 1, last), bufs[1])
    drain(i0, bufs[0])
    compute(i0, bufs[0])
    fire(jnp.minimum(i0 + 2, last), bufs[0])
    drain(jnp.minimum(i0 + 1, last), bufs[1])
    compute(i0 + 1, bufs[1])
    return 0

  lax.fori_loop(0, NCHUNK // 2, pair, 0)
  drain(last, bufs[0])

  pltpu.sync_copy(score, out.at[pl.ds(wid * (NNEG + 1), NNEG + 1)])


_sc_scores = functools.partial(
    pl.kernel,
    out_type=jax.ShapeDtypeStruct((NW * (NNEG + 1), BPW), jnp.float32),
    mesh=plsc.VectorSubcoreMesh(core_axis_name="c", subcore_axis_name="s"),
    compiler_params=pltpu.CompilerParams(
        needs_layout_passes=False, use_tc_tiling_on_sc=False),
    scratch_types=[
        pltpu.VMEM((BPW,), jnp.int32),
        pltpu.VMEM((BPW,), jnp.int32),
        pltpu.VMEM((BPW * NNEG,), jnp.int32),
        pltpu.VMEM((BPW,), jnp.int32),
        pltpu.VMEM((BPW,), jnp.int32),
        pltpu.VMEM((NNEG, BPW), jnp.int32),
        pltpu.VMEM((CB, PDIM), jnp.float32),
        pltpu.VMEM((CB, PDIM), jnp.float32),
        pltpu.VMEM((NEG_PER_CHUNK, PDIM), jnp.float32),
        pltpu.VMEM((CB, PDIM), jnp.float32),
        pltpu.VMEM((CB, PDIM), jnp.float32),
        pltpu.VMEM((NEG_PER_CHUNK, PDIM), jnp.float32),
        pltpu.VMEM((NNEG + 1, BPW), jnp.float32),
        pltpu.SemaphoreType.DMA,
        pltpu.SemaphoreType.DMA,
    ],
)(_sc_scores_body)


TBK = 12800
NTBK = 40
PROWS = NTBK * TBK


def _transpose_body(a_ref, b_ref, o_ref):
  o_ref[:, 0:DIM] = a_ref[...].T
  o_ref[:, DIM:2 * DIM] = b_ref[...].T


_repack = pl.pallas_call(
    _transpose_body,
    grid=(NTBK,),
    in_specs=[
        pl.BlockSpec((DIM, TBK), lambda g: (0, 2 * g)),
        pl.BlockSpec((DIM, TBK),
                     lambda g: (0, jnp.minimum(2 * g + 1, VOCAB // TBK))),
    ],
    out_specs=pl.BlockSpec((TBK, PDIM), lambda g: (g, 0)),
    out_shape=jax.ShapeDtypeStruct((PROWS, PDIM), jnp.float32),
)


def _repack_table(table):
  tt = table.T
  return _repack(tt, tt)


def _loss_body(s_ref, o_ref):
  x = s_ref[...]
  y = jnp.minimum(x, 0.0) - jnp.log1p(jnp.exp(-jnp.abs(x)))
  o_ref[0, 0] = -jnp.sum(y) * (1.0 / BATCH)


_loss = pl.pallas_call(
    _loss_body,
    out_shape=jax.ShapeDtypeStruct((1, 1), jnp.float32),
    out_specs=pl.BlockSpec(memory_space=pltpu.SMEM),
)


def kernel(center_ids, context_ids, negative_ids, center_table, context_table):
  cen_ids = center_ids.astype(jnp.int32)
  ctx_ids = context_ids.astype(jnp.int32)
  neg_ids = negative_ids.astype(jnp.int32)

  def row(ids):
    blk = ids // TBK
    return (blk >> 1) * TBK + ids % TBK

  def colb(ids):
    return ((ids // TBK) & 1) << 6

  scores = _sc_scores(
      row(cen_ids), row(ctx_ids), row(neg_ids).reshape(BATCH * NNEG),
      colb(cen_ids), colb(ctx_ids), colb(neg_ids).T,
      _repack_table(center_table), _repack_table(context_table))
  return _loss(scores)[0, 0]


# speedup vs baseline: 2.2745x; 1.1606x over previous
import functools

import jax
import jax.numpy as jnp
from jax import lax
from jax.experimental import pallas as pl
from jax.experimental.pallas import tpu as pltpu
from jax.experimental.pallas import tpu_sc as plsc

VOCAB = 1000000
DIM = 64
BATCH = 16384
NNEG = 20

NC = 2
NS = 16
NW = NC * NS
BPW = BATCH // NW
CB = 16
NCHUNK = BPW // CB
NEG_SEG = 64
NEG_PER_CHUNK = CB * NNEG
NEG_DMAS = NEG_PER_CHUNK // NEG_SEG
PDIM = 2 * DIM


def _sc_scores_body(cenp, ctxp, negp, cenc, ctxc, negc, cen_tab, ctx_tab, out,
                    idxc, idxx, idxn, ccolb, xcolb, ncolb, cenv0, ctxv0,
                    negv0, cenv1, ctxv1, negv1, score, sem0, sem1):
  wid = lax.axis_index("s") * NC + lax.axis_index("c")
  base = wid * BPW

  pltpu.sync_copy(cenp.at[pl.ds(base, BPW)], idxc)
  pltpu.sync_copy(ctxp.at[pl.ds(base, BPW)], idxx)
  pltpu.sync_copy(negp.at[pl.ds(base * NNEG, BPW * NNEG)], idxn)
  pltpu.sync_copy(cenc.at[pl.ds(base, BPW)], ccolb)
  pltpu.sync_copy(ctxc.at[pl.ds(base, BPW)], xcolb)
  pltpu.sync_copy(negc.at[:, pl.ds(base, BPW)], ncolb)

  iota16 = lax.iota(jnp.int32, 16)
  lane15 = iota16 == 15
  nsplat = [jnp.full((16,), n, jnp.int32) for n in range(NNEG + 1)]
  bufs = ((cenv0, ctxv0, negv0, sem0), (cenv1, ctxv1, negv1, sem1))
  last = NCHUNK - 1

  def descs(i, buf):
    cenv, ctxv, negv, sem = buf
    ds = [
        pltpu.make_async_copy(cen_tab.at[idxc.at[pl.ds(i * CB, CB)]],
                              cenv, sem),
        pltpu.make_async_copy(ctx_tab.at[idxx.at[pl.ds(i * CB, CB)]],
                              ctxv, sem),
    ]
    for j in range(NEG_DMAS):
      ds.append(
          pltpu.make_async_copy(
              ctx_tab.at[idxn.at[pl.ds(i * NEG_PER_CHUNK + j * NEG_SEG,
                                       NEG_SEG)]],
              negv.at[pl.ds(j * NEG_SEG, NEG_SEG)], sem))
    return ds

  def fire(i, buf):
    for d in descs(i, buf):
      d.start()

  def drain(i, buf):
    for d in descs(i, buf):
      d.wait()

  def compute(i, buf):
    cenv, ctxv, negv, _ = buf

    def body(b, _):
      brow = jnp.broadcast_to(b, (16,)).astype(jnp.int32)
      col = jnp.broadcast_to(i * CB + b, (16,))

      ccol = plsc.load_gather(ccolb, [col]) + iota16
      xcol = plsc.load_gather(xcolb, [col]) + iota16
      cs = [plsc.load_gather(cenv, [brow, ccol + 16 * k]) for k in range(4)]
      xs = [plsc.load_gather(ctxv, [brow, xcol + 16 * k]) for k in range(4)]
      p = cs[0] * xs[0] + cs[1] * xs[1] + cs[2] * xs[2] + cs[3] * xs[3]
      plsc.store_scatter(score, [nsplat[NNEG], col], plsc.cumsum(p),
                         mask=lane15)
      for n in range(NNEG):
        nrow = jnp.broadcast_to(b * NNEG + n, (16,)).astype(jnp.int32)
        gcol = plsc.load_gather(ncolb, [nsplat[n], col]) + iota16
        gs = [plsc.load_gather(negv, [nrow, gcol + 16 * k]) for k in range(4)]
        q = cs[0] * gs[0] + cs[1] * gs[1] + cs[2] * gs[2] + cs[3] * gs[3]
        plsc.store_scatter(score, [nsplat[n], col], -plsc.cumsum(q),
                           mask=lane15)
      return 0

    lax.fori_loop(0, CB, body, 0)

  fire(0, bufs[0])

  def pair(ip, _):
    i0 = 2 * ip
    fire(jnp.minimum(i0 + 1, last), bufs[1])
    drain(i0, bufs[0])
    compute(i0, bufs[0])
    fire(jnp.minimum(i0 + 2, last), bufs[0])
    drain(jnp.minimum(i0 + 1, last), bufs[1])
    compute(i0 + 1, bufs[1])
    return 0

  lax.fori_loop(0, NCHUNK // 2, pair, 0)
  drain(last, bufs[0])

  pltpu.sync_copy(score, out.at[pl.ds(wid * (NNEG + 1), NNEG + 1)])


_sc_scores = functools.partial(
    pl.kernel,
    out_type=jax.ShapeDtypeStruct((NW * (NNEG + 1), BPW), jnp.float32),
    mesh=plsc.VectorSubcoreMesh(core_axis_name="c", subcore_axis_name="s"),
    compiler_params=pltpu.CompilerParams(
        needs_layout_passes=False, use_tc_tiling_on_sc=False),
    scratch_types=[
        pltpu.VMEM((BPW,), jnp.int32),
        pltpu.VMEM((BPW,), jnp.int32),
        pltpu.VMEM((BPW * NNEG,), jnp.int32),
        pltpu.VMEM((BPW,), jnp.int32),
        pltpu.VMEM((BPW,), jnp.int32),
        pltpu.VMEM((NNEG, BPW), jnp.int32),
        pltpu.VMEM((CB, PDIM), jnp.float32),
        pltpu.VMEM((CB, PDIM), jnp.float32),
        pltpu.VMEM((NEG_PER_CHUNK, PDIM), jnp.float32),
        pltpu.VMEM((CB, PDIM), jnp.float32),
        pltpu.VMEM((CB, PDIM), jnp.float32),
        pltpu.VMEM((NEG_PER_CHUNK, PDIM), jnp.float32),
        pltpu.VMEM((NNEG + 1, BPW), jnp.float32),
        pltpu.SemaphoreType.DMA,
        pltpu.SemaphoreType.DMA,
    ],
)(_sc_scores_body)


TBK = 12800
NTBK = 40
PROWS = NTBK * TBK


def _transpose_body(a_ref, b_ref, o_ref):
  eye = (lax.broadcasted_iota(jnp.int32, (PDIM, PDIM), 0) ==
         lax.broadcasted_iota(jnp.int32, (PDIM, PDIM), 1)).astype(jnp.float32)
  ab = jnp.concatenate([a_ref[...], b_ref[...]], axis=0)
  o_ref[...] = lax.dot_general(ab, eye, (((0,), (0,)), ((), ())))


_repack = pl.pallas_call(
    _transpose_body,
    grid=(NTBK,),
    in_specs=[
        pl.BlockSpec((DIM, TBK), lambda g: (0, 2 * g)),
        pl.BlockSpec((DIM, TBK),
                     lambda g: (0, jnp.minimum(2 * g + 1, VOCAB // TBK))),
    ],
    out_specs=pl.BlockSpec((TBK, PDIM), lambda g: (g, 0)),
    out_shape=jax.ShapeDtypeStruct((PROWS, PDIM), jnp.float32),
)


def _repack_table(table):
  tt = table.T
  return _repack(tt, tt)


def _loss_body(s_ref, o_ref):
  x = s_ref[...]
  y = jnp.minimum(x, 0.0) - jnp.log1p(jnp.exp(-jnp.abs(x)))
  o_ref[0, 0] = -jnp.sum(y) * (1.0 / BATCH)


_loss = pl.pallas_call(
    _loss_body,
    out_shape=jax.ShapeDtypeStruct((1, 1), jnp.float32),
    out_specs=pl.BlockSpec(memory_space=pltpu.SMEM),
)


def kernel(center_ids, context_ids, negative_ids, center_table, context_table):
  cen_ids = center_ids.astype(jnp.int32)
  ctx_ids = context_ids.astype(jnp.int32)
  neg_ids = negative_ids.astype(jnp.int32)

  def row(ids):
    blk = ids // TBK
    return (blk >> 1) * TBK + ids % TBK

  def colb(ids):
    return ((ids // TBK) & 1) << 6

  scores = _sc_scores(
      row(cen_ids), row(ctx_ids), row(neg_ids).reshape(BATCH * NNEG),
      colb(cen_ids), colb(ctx_ids), colb(neg_ids).T,
      _repack_table(center_table), _repack_table(context_table))
  return _loss(scores)[0, 0]


# speedup vs baseline: 2.2788x; 1.0019x over previous
import functools

import jax
import jax.numpy as jnp
from jax import lax
from jax.experimental import pallas as pl
from jax.experimental.pallas import tpu as pltpu
from jax.experimental.pallas import tpu_sc as plsc

VOCAB = 1000000
DIM = 64
BATCH = 16384
NNEG = 20

NC = 2
NS = 16
NW = NC * NS
BPW = BATCH // NW
CB = 16
NCHUNK = BPW // CB
ROWS_PER_CHUNK = CB * (NNEG + 2)
CXN = CB * (NNEG + 1)
GSEGS = (128, 128, 80)
IPW = NCHUNK * CXN
PDIM = 2 * DIM


def _sc_scores_body(cenp, ids_cat, cenc, ctxc, negc, cen_tab, ctx_tab, out,
                    idxc, idxa, ccolb, xcolb, ncolb, gbuf0, gbuf1, score,
                    sem0, sem1):
  wid = lax.axis_index("s") * NC + lax.axis_index("c")
  base = wid * BPW

  pltpu.sync_copy(cenp.at[pl.ds(base, BPW)], idxc)
  pltpu.sync_copy(ids_cat.at[pl.ds(wid * IPW, IPW)], idxa)
  pltpu.sync_copy(cenc.at[pl.ds(base, BPW)], ccolb)
  pltpu.sync_copy(ctxc.at[pl.ds(base, BPW)], xcolb)
  pltpu.sync_copy(negc.at[:, pl.ds(base, BPW)], ncolb)

  iota16 = lax.iota(jnp.int32, 16)
  lane15 = iota16 == 15
  nsplat = [jnp.full((16,), n, jnp.int32) for n in range(NNEG + 1)]
  bufs = ((gbuf0, sem0), (gbuf1, sem1))
  last = NCHUNK - 1

  def descs(i, buf):
    gbuf, sem = buf
    ds = [
        pltpu.make_async_copy(cen_tab.at[idxc.at[pl.ds(i * CB, CB)]],
                              gbuf.at[pl.ds(0, CB)], sem),
    ]
    off = 0
    for seg in GSEGS:
      ds.append(
          pltpu.make_async_copy(
              ctx_tab.at[idxa.at[pl.ds(i * CXN + off, seg)]],
              gbuf.at[pl.ds(CB + off, seg)], sem))
      off += seg
    return ds

  def fire(i, buf):
    for d in descs(i, buf):
      d.start()

  def drain(i, buf):
    for d in descs(i, buf):
      d.wait()

  def compute(i, buf):
    gbuf, _ = buf

    def body(b, _):
      brow = jnp.broadcast_to(b, (16,)).astype(jnp.int32)
      col = jnp.broadcast_to(i * CB + b, (16,))

      ccol = plsc.load_gather(ccolb, [col]) + iota16
      xcol = plsc.load_gather(xcolb, [col]) + iota16
      cs = [plsc.load_gather(gbuf, [brow, ccol + 16 * k]) for k in range(4)]
      xs = [plsc.load_gather(gbuf, [brow + CB, xcol + 16 * k])
            for k in range(4)]
      p = cs[0] * xs[0] + cs[1] * xs[1] + cs[2] * xs[2] + cs[3] * xs[3]
      plsc.store_scatter(score, [nsplat[NNEG], col], plsc.cumsum(p),
                         mask=lane15)
      for n in range(NNEG):
        nrow = jnp.broadcast_to(2 * CB + b * NNEG + n, (16,)).astype(jnp.int32)
        gcol = plsc.load_gather(ncolb, [nsplat[n], col]) + iota16
        gs = [plsc.load_gather(gbuf, [nrow, gcol + 16 * k]) for k in range(4)]
        q = cs[0] * gs[0] + cs[1] * gs[1] + cs[2] * gs[2] + cs[3] * gs[3]
        plsc.store_scatter(score, [nsplat[n], col], -plsc.cumsum(q),
                           mask=lane15)
      return 0

    lax.fori_loop(0, CB, body, 0)

  fire(0, bufs[0])

  def pair(ip, _):
    i0 = 2 * ip
    fire(jnp.minimum(i0 + 1, last), bufs[1])
    drain(i0, bufs[0])
    compute(i0, bufs[0])
    fire(jnp.minimum(i0 + 2, last), bufs[0])
    drain(jnp.minimum(i0 + 1, last), bufs[1])
    compute(i0 + 1, bufs[1])
    return 0

  lax.fori_loop(0, NCHUNK // 2, pair, 0)
  drain(last, bufs[0])

  pltpu.sync_copy(score, out.at[pl.ds(wid * (NNEG + 1), NNEG + 1)])


_sc_scores = functools.partial(
    pl.kernel,
    out_type=jax.ShapeDtypeStruct((NW * (NNEG + 1), BPW), jnp.float32),
    mesh=plsc.VectorSubcoreMesh(core_axis_name="c", subcore_axis_name="s"),
    compiler_params=pltpu.CompilerParams(
        needs_layout_passes=False, use_tc_tiling_on_sc=False),
    scratch_types=[
        pltpu.VMEM((BPW,), jnp.int32),
        pltpu.VMEM((IPW,), jnp.int32),
        pltpu.VMEM((BPW,), jnp.int32),
        pltpu.VMEM((BPW,), jnp.int32),
        pltpu.VMEM((NNEG, BPW), jnp.int32),
        pltpu.VMEM((ROWS_PER_CHUNK, PDIM), jnp.float32),
        pltpu.VMEM((ROWS_PER_CHUNK, PDIM), jnp.float32),
        pltpu.VMEM((NNEG + 1, BPW), jnp.float32),
        pltpu.SemaphoreType.DMA,
        pltpu.SemaphoreType.DMA,
    ],
)(_sc_scores_body)


TBK = 12800
NTBK = 40
PROWS = NTBK * TBK


def _transpose_body(a_ref, b_ref, o_ref):
  eye = (lax.broadcasted_iota(jnp.int32, (PDIM, PDIM), 0) ==
         lax.broadcasted_iota(jnp.int32, (PDIM, PDIM), 1)).astype(jnp.float32)
  ab = jnp.concatenate([a_ref[...], b_ref[...]], axis=0)
  o_ref[...] = lax.dot_general(ab, eye, (((0,), (0,)), ((), ())))


_repack = pl.pallas_call(
    _transpose_body,
    grid=(NTBK,),
    in_specs=[
        pl.BlockSpec((DIM, TBK), lambda g: (0, 2 * g)),
        pl.BlockSpec((DIM, TBK),
                     lambda g: (0, jnp.minimum(2 * g + 1, VOCAB // TBK))),
    ],
    out_specs=pl.BlockSpec((TBK, PDIM), lambda g: (g, 0)),
    out_shape=jax.ShapeDtypeStruct((PROWS, PDIM), jnp.float32),
)


def _repack_table(table):
  tt = table.T
  return _repack(tt, tt)


def _loss_body(s_ref, o_ref):
  x = s_ref[...]
  y = jnp.minimum(x, 0.0) - jnp.log1p(jnp.exp(-jnp.abs(x)))
  o_ref[0, 0] = -jnp.sum(y) * (1.0 / BATCH)


_loss = pl.pallas_call(
    _loss_body,
    out_shape=jax.ShapeDtypeStruct((1, 1), jnp.float32),
    out_specs=pl.BlockSpec(memory_space=pltpu.SMEM),
)


def kernel(center_ids, context_ids, negative_ids, center_table, context_table):
  cen_ids = center_ids.astype(jnp.int32)
  ctx_ids = context_ids.astype(jnp.int32)
  neg_ids = negative_ids.astype(jnp.int32)

  def row(ids):
    blk = ids // TBK
    return (blk >> 1) * TBK + ids % TBK

  def colb(ids):
    return ((ids // TBK) & 1) << 6

  ids_cat = jnp.concatenate(
      [row(ctx_ids).reshape(NW, NCHUNK, CB),
       row(neg_ids).reshape(NW, NCHUNK, CB * NNEG)],
      axis=2).reshape(NW * IPW)
  scores = _sc_scores(
      row(cen_ids), ids_cat,
      colb(cen_ids), colb(ctx_ids), colb(neg_ids).T,
      _repack_table(center_table), _repack_table(context_table))
  return _loss(scores)[0, 0]
